# baseline (speedup 1.0000x reference)
import functools

import jax
import jax.numpy as jnp
from jax import lax
from jax.experimental import pallas as pl
from jax.experimental.pallas import tpu as pltpu
from jax.experimental.pallas import tpu_sc as plsc

_B = 16384
_L = 200
_D = 64
_V = 1000000
_NC = 2
_NS = 16
_NW = _NC * _NS
_RPT = _B // _NW
_SPLIT = 128
_REST = _L - _SPLIT
_TC = 128


def _sigmoid(x):
    return 1.0 / (1.0 + jnp.exp(-x))


def _tr_body(x_ref, o_ref):
    X = x_ref[...]
    c = lax.broadcasted_iota(jnp.int32, (_TC, _TC // 2), 0)
    j = lax.broadcasted_iota(jnp.int32, (_TC, _TC // 2), 1)
    Ee = (c == 2 * j).astype(jnp.float32)
    Eo = (c == 2 * j + 1).astype(jnp.float32)
    dn = (((0,), (1,)), ((), ()))
    Ye = lax.dot_general(Ee, X, dn, preferred_element_type=jnp.float32,
                         precision=lax.Precision.HIGHEST)
    Yo = lax.dot_general(Eo, X, dn, preferred_element_type=jnp.float32,
                         precision=lax.Precision.HIGHEST)
    o_ref[:, 0:64] = Ye
    o_ref[:, 64:128] = Yo


def _to_row_major(table):
    tpairs = pl.pallas_call(
        _tr_body,
        grid=((_V + _TC - 1) // _TC,),
        in_specs=[pl.BlockSpec((_D, _TC), lambda i: (0, i))],
        out_specs=pl.BlockSpec((_TC // 2, 128), lambda i: (i, 0)),
        out_shape=jax.ShapeDtypeStruct((_V // 2, 128), jnp.float32),
    )(table.T)
    return tpairs.reshape(_V, _D)


def _body(idx_hbm, wb_hbm, table_hbm, out_hbm, idx_v, buf0, buf1, wb_v,
          out_v, out_smem, sem0, sem1):
    wid = lax.axis_index("s") * _NC + lax.axis_index("c")
    base = wid * _RPT

    pltpu.sync_copy(wb_hbm, wb_v)
    pltpu.sync_copy(idx_hbm.at[pl.ds(base, _RPT)], idx_v)

    w0 = wb_v[pl.ds(0, 16)]
    w1 = wb_v[pl.ds(16, 16)]
    w2 = wb_v[pl.ds(32, 16)]
    w3 = wb_v[pl.ds(48, 16)]
    bvec = wb_v[pl.ds(64, 16)]

    def fire(r, buf, sem):
        pltpu.async_copy(
            table_hbm.at[idx_v.at[r, pl.ds(0, _SPLIT)]],
            buf.at[pl.ds(0, _SPLIT)], sem)
        pltpu.async_copy(
            table_hbm.at[idx_v.at[r, pl.ds(_SPLIT, _REST)]],
            buf.at[pl.ds(_SPLIT, _REST)], sem)

    def wait(buf, sem):
        pltpu.make_async_copy(table_hbm.at[pl.ds(0, _L)], buf, sem).wait()

    def process(r, buf):
        def red(j, accs):
            a0, a1, a2, a3 = accs
            row = buf.at[j]
            return (a0 + row[pl.ds(0, 16)],
                    a1 + row[pl.ds(16, 16)],
                    a2 + row[pl.ds(32, 16)],
                    a3 + row[pl.ds(48, 16)])

        z = jnp.zeros((16,), jnp.float32)
        a0, a1, a2, a3 = lax.fori_loop(0, _L, red, (z, z, z, z), unroll=8)
        inv = jnp.float32(1.0 / _L)
        h0 = _sigmoid(a0 * inv)
        h1 = _sigmoid(a1 * inv)
        h2 = _sigmoid(a2 * inv)
        h3 = _sigmoid(a3 * inv)
        t = h0 * w0 + h1 * w1 + h2 * w2 + h3 * w3 + bvec
        out_smem[r] = jnp.sum(t)

    fire(0, buf0, sem0)

    def loop(i, carry):
        r0 = 2 * i
        fire(r0 + 1, buf1, sem1)
        wait(buf0, sem0)
        process(r0, buf0)

        @pl.when(r0 + 2 < _RPT)
        def _():
            fire(r0 + 2, buf0, sem0)

        wait(buf1, sem1)
        process(r0 + 1, buf1)
        return carry

    lax.fori_loop(0, _RPT // 2, loop, 0)

    lane = lax.iota(jnp.int32, 16)

    def pack(g, carry):
        def ins(k, v):
            return jnp.where(lane == k, out_smem[g * 16 + k], v)

        v = lax.fori_loop(0, 16, ins, jnp.zeros((16,), jnp.float32))
        out_v[pl.ds(g * 16, 16)] = _sigmoid(v)
        return carry

    lax.fori_loop(0, _RPT // 16, pack, 0)

    pltpu.sync_copy(out_v, out_hbm.at[pl.ds(base, _RPT)])


@jax.jit
def _run(idx, wb, table):
    table = _to_row_major(table)
    mesh = plsc.VectorSubcoreMesh(core_axis_name="c", subcore_axis_name="s")
    f = pl.kernel(
        _body,
        out_type=jax.ShapeDtypeStruct((_B,), jnp.float32),
        mesh=mesh,
        compiler_params=pltpu.CompilerParams(
            needs_layout_passes=False, use_tc_tiling_on_sc=False),
        scratch_types=[
            pltpu.VMEM((_RPT, _L), jnp.int32),
            pltpu.VMEM((_L, _D), jnp.float32),
            pltpu.VMEM((_L, _D), jnp.float32),
            pltpu.VMEM((80,), jnp.float32),
            pltpu.VMEM((_RPT,), jnp.float32),
            pltpu.SMEM((_RPT,), jnp.float32),
            pltpu.SemaphoreType.DMA,
            pltpu.SemaphoreType.DMA,
        ],
    )
    return f(idx, wb, table)


def kernel(input_ids, emb_table, W, b):
    wb = jnp.concatenate(
        [W.reshape(-1), b.reshape(-1),
         jnp.zeros((15,), jnp.float32)]).astype(jnp.float32)
    out = _run(input_ids, wb, emb_table)
    return out.reshape(_B, 1)


# speedup vs baseline: 3.7670x; 3.7670x over previous
import functools

import jax
import jax.numpy as jnp
from jax import lax
from jax.experimental import pallas as pl
from jax.experimental.pallas import tpu as pltpu
from jax.experimental.pallas import tpu_sc as plsc

_B = 16384
_L = 200
_D = 64
_V = 1000000
_NC = 2
_NS = 16
_NW = _NC * _NS
_RPT = _B // _NW
_SPLIT = 128
_REST = _L - _SPLIT
_TC = 2048
_TCC = 128


def _sigmoid(x):
    return 1.0 / (1.0 + jnp.exp(-x))


def _tr_body(x_ref, o_ref):
    c = lax.broadcasted_iota(jnp.int32, (_TCC, _TCC // 2), 0)
    j = lax.broadcasted_iota(jnp.int32, (_TCC, _TCC // 2), 1)
    Ee = (c == 2 * j).astype(jnp.float32)
    Eo = (c == 2 * j + 1).astype(jnp.float32)
    dn = (((0,), (1,)), ((), ()))
    for k in range(_TC // _TCC):
        X = x_ref[:, k * _TCC:(k + 1) * _TCC]
        Ye = lax.dot_general(Ee, X, dn, preferred_element_type=jnp.float32,
                             precision=lax.Precision.HIGHEST)
        Yo = lax.dot_general(Eo, X, dn, preferred_element_type=jnp.float32,
                             precision=lax.Precision.HIGHEST)
        r = k * (_TCC // 2)
        o_ref[r:r + _TCC // 2, 0:64] = Ye
        o_ref[r:r + _TCC // 2, 64:128] = Yo


def _to_row_major(table):
    tpairs = pl.pallas_call(
        _tr_body,
        grid=((_V + _TC - 1) // _TC,),
        in_specs=[pl.BlockSpec((_D, _TC), lambda i: (0, i))],
        out_specs=pl.BlockSpec((_TC // 2, 128), lambda i: (i, 0)),
        out_shape=jax.ShapeDtypeStruct((_V // 2, 128), jnp.float32),
    )(table.T)
    return tpairs.reshape(_V, _D)


def _body(idx_hbm, wb_hbm, table_hbm, out_hbm, idx_v, buf0, buf1, wb_v,
          out_v, out_smem, sem0, sem1):
    wid = lax.axis_index("s") * _NC + lax.axis_index("c")
    base = wid * _RPT

    pltpu.sync_copy(wb_hbm, wb_v)
    pltpu.sync_copy(idx_hbm.at[pl.ds(base, _RPT)], idx_v)

    w0 = wb_v[pl.ds(0, 16)]
    w1 = wb_v[pl.ds(16, 16)]
    w2 = wb_v[pl.ds(32, 16)]
    w3 = wb_v[pl.ds(48, 16)]
    bvec = wb_v[pl.ds(64, 16)]

    def fire(r, buf, sem):
        pltpu.async_copy(
            table_hbm.at[idx_v.at[r, pl.ds(0, _SPLIT)]],
            buf.at[pl.ds(0, _SPLIT)], sem)
        pltpu.async_copy(
            table_hbm.at[idx_v.at[r, pl.ds(_SPLIT, _REST)]],
            buf.at[pl.ds(_SPLIT, _REST)], sem)

    def wait(buf, sem):
        pltpu.make_async_copy(table_hbm.at[pl.ds(0, _L)], buf, sem).wait()

    def process(r, buf):
        def red(j, accs):
            a0, a1, a2, a3 = accs
            row = buf.at[j]
            return (a0 + row[pl.ds(0, 16)],
                    a1 + row[pl.ds(16, 16)],
                    a2 + row[pl.ds(32, 16)],
                    a3 + row[pl.ds(48, 16)])

        z = jnp.zeros((16,), jnp.float32)
        a0, a1, a2, a3 = lax.fori_loop(0, _L, red, (z, z, z, z), unroll=8)
        inv = jnp.float32(1.0 / _L)
        h0 = _sigmoid(a0 * inv)
        h1 = _sigmoid(a1 * inv)
        h2 = _sigmoid(a2 * inv)
        h3 = _sigmoid(a3 * inv)
        t = h0 * w0 + h1 * w1 + h2 * w2 + h3 * w3 + bvec
        out_smem[r] = jnp.sum(t)

    fire(0, buf0, sem0)

    def loop(i, carry):
        r0 = 2 * i
        fire(r0 + 1, buf1, sem1)
        wait(buf0, sem0)
        process(r0, buf0)

        @pl.when(r0 + 2 < _RPT)
        def _():
            fire(r0 + 2, buf0, sem0)

        wait(buf1, sem1)
        process(r0 + 1, buf1)
        return carry

    lax.fori_loop(0, _RPT // 2, loop, 0)

    lane = lax.iota(jnp.int32, 16)

    def pack(g, carry):
        def ins(k, v):
            return jnp.where(lane == k, out_smem[g * 16 + k], v)

        v = lax.fori_loop(0, 16, ins, jnp.zeros((16,), jnp.float32))
        out_v[pl.ds(g * 16, 16)] = _sigmoid(v)
        return carry

    lax.fori_loop(0, _RPT // 16, pack, 0)

    pltpu.sync_copy(out_v, out_hbm.at[pl.ds(base, _RPT)])


@jax.jit
def _run(idx, wb, table):
    table = _to_row_major(table)
    mesh = plsc.VectorSubcoreMesh(core_axis_name="c", subcore_axis_name="s")
    f = pl.kernel(
        _body,
        out_type=jax.ShapeDtypeStruct((_B,), jnp.float32),
        mesh=mesh,
        compiler_params=pltpu.CompilerParams(
            needs_layout_passes=False, use_tc_tiling_on_sc=False),
        scratch_types=[
            pltpu.VMEM((_RPT, _L), jnp.int32),
            pltpu.VMEM((_L, _D), jnp.float32),
            pltpu.VMEM((_L, _D), jnp.float32),
            pltpu.VMEM((80,), jnp.float32),
            pltpu.VMEM((_RPT,), jnp.float32),
            pltpu.SMEM((_RPT,), jnp.float32),
            pltpu.SemaphoreType.DMA,
            pltpu.SemaphoreType.DMA,
        ],
    )
    return f(idx, wb, table)


def kernel(input_ids, emb_table, W, b):
    wb = jnp.concatenate(
        [W.reshape(-1), b.reshape(-1),
         jnp.zeros((15,), jnp.float32)]).astype(jnp.float32)
    out = _run(input_ids, wb, emb_table)
    return out.reshape(_B, 1)


# speedup vs baseline: 6.8449x; 1.8171x over previous
import functools

import jax
import jax.numpy as jnp
from jax import lax
from jax.experimental import pallas as pl
from jax.experimental.pallas import tpu as pltpu
from jax.experimental.pallas import tpu_sc as plsc

_B = 16384
_L = 200
_D = 64
_V = 1000000
_NC = 2
_NS = 16
_NW = _NC * _NS
_RPT = _B // _NW
_SPLIT = 128
_REST = _L - _SPLIT
_TC = 4096
_NBLK = 122
_A = _NBLK * _TC
_TAILB = 640
_PR = _A + _TAILB


def _sigmoid(x):
    return 1.0 / (1.0 + jnp.exp(-x))


def _tr_body(xlo_ref, xhi_ref, o_ref):
    o_ref[:, 0:_D] = xlo_ref[...].T
    o_ref[:, _D:2 * _D] = xhi_ref[...].T


def _tail_body(x_ref, alias_ref, o_ref):
    del alias_ref
    o_ref[:, 0:_D] = jnp.zeros((128, _D), jnp.float32)
    o_ref[:, _D:2 * _D] = x_ref[...].T


def _to_row_major(table):
    tpairs = pl.pallas_call(
        _tr_body,
        grid=(_NBLK,),
        in_specs=[pl.BlockSpec((_D, _TC), lambda i: (0, i)),
                  pl.BlockSpec((_D, _TC), lambda i: (0, i + _NBLK))],
        out_specs=pl.BlockSpec((_TC, 2 * _D), lambda i: (i, 0)),
        out_shape=jax.ShapeDtypeStruct((_PR, 2 * _D), jnp.float32),
    )(table.T, table.T)
    tpairs = pl.pallas_call(
        _tail_body,
        grid=(_TAILB // 128,),
        in_specs=[pl.BlockSpec((_D, 128), lambda i: (0, 2 * _A // 128 + i)),
                  pl.BlockSpec(memory_space=pl.ANY)],
        out_specs=pl.BlockSpec((128, 2 * _D), lambda i: (_A // 128 + i, 0)),
        out_shape=jax.ShapeDtypeStruct((_PR, 2 * _D), jnp.float32),
        input_output_aliases={1: 0},
    )(table.T, tpairs)
    return tpairs.reshape(2 * _PR, _D)


def _remap_body(i_ref, o_ref):
    v = i_ref[...]
    o_ref[...] = jnp.where(v < _A, 2 * v, 2 * (v - _A) + 1)


def _remap_ids(ids):
    blk = 2048
    return pl.pallas_call(
        _remap_body,
        grid=(_B // blk,),
        in_specs=[pl.BlockSpec((blk, _L), lambda i: (i, 0))],
        out_specs=pl.BlockSpec((blk, _L), lambda i: (i, 0)),
        out_shape=jax.ShapeDtypeStruct((_B, _L), jnp.int32),
    )(ids)


def _body(idx_hbm, wb_hbm, table_hbm, out_hbm, idx_v, buf0, buf1, wb_v,
          out_v, out_smem, sem0, sem1):
    wid = lax.axis_index("s") * _NC + lax.axis_index("c")
    base = wid * _RPT

    pltpu.sync_copy(wb_hbm, wb_v)
    pltpu.sync_copy(idx_hbm.at[pl.ds(base, _RPT)], idx_v)

    w0 = wb_v[pl.ds(0, 16)]
    w1 = wb_v[pl.ds(16, 16)]
    w2 = wb_v[pl.ds(32, 16)]
    w3 = wb_v[pl.ds(48, 16)]
    bvec = wb_v[pl.ds(64, 16)]

    def fire(r, buf, sem):
        pltpu.async_copy(
            table_hbm.at[idx_v.at[r, pl.ds(0, _SPLIT)]],
            buf.at[pl.ds(0, _SPLIT)], sem)
        pltpu.async_copy(
            table_hbm.at[idx_v.at[r, pl.ds(_SPLIT, _REST)]],
            buf.at[pl.ds(_SPLIT, _REST)], sem)

    def wait(buf, sem):
        pltpu.make_async_copy(table_hbm.at[pl.ds(0, _L)], buf, sem).wait()

    def process(r, buf):
        def red(j, accs):
            a0, a1, a2, a3 = accs
            row = buf.at[j]
            return (a0 + row[pl.ds(0, 16)],
                    a1 + row[pl.ds(16, 16)],
                    a2 + row[pl.ds(32, 16)],
                    a3 + row[pl.ds(48, 16)])

        z = jnp.zeros((16,), jnp.float32)
        a0, a1, a2, a3 = lax.fori_loop(0, _L, red, (z, z, z, z), unroll=8)
        inv = jnp.float32(1.0 / _L)
        h0 = _sigmoid(a0 * inv)
        h1 = _sigmoid(a1 * inv)
        h2 = _sigmoid(a2 * inv)
        h3 = _sigmoid(a3 * inv)
        t = h0 * w0 + h1 * w1 + h2 * w2 + h3 * w3 + bvec
        out_smem[r] = jnp.sum(t)

    fire(0, buf0, sem0)

    def loop(i, carry):
        r0 = 2 * i
        fire(r0 + 1, buf1, sem1)
        wait(buf0, sem0)
        process(r0, buf0)

        @pl.when(r0 + 2 < _RPT)
        def _():
            fire(r0 + 2, buf0, sem0)

        wait(buf1, sem1)
        process(r0 + 1, buf1)
        return carry

    lax.fori_loop(0, _RPT // 2, loop, 0)

    lane = lax.iota(jnp.int32, 16)

    def pack(g, carry):
        def ins(k, v):
            return jnp.where(lane == k, out_smem[g * 16 + k], v)

        v = lax.fori_loop(0, 16, ins, jnp.zeros((16,), jnp.float32))
        out_v[pl.ds(g * 16, 16)] = _sigmoid(v)
        return carry

    lax.fori_loop(0, _RPT // 16, pack, 0)

    pltpu.sync_copy(out_v, out_hbm.at[pl.ds(base, _RPT)])


@jax.jit
def _run(idx, wb, table):
    table = _to_row_major(table)
    idx = _remap_ids(idx)
    mesh = plsc.VectorSubcoreMesh(core_axis_name="c", subcore_axis_name="s")
    f = pl.kernel(
        _body,
        out_type=jax.ShapeDtypeStruct((_B,), jnp.float32),
        mesh=mesh,
        compiler_params=pltpu.CompilerParams(
            needs_layout_passes=False, use_tc_tiling_on_sc=False),
        scratch_types=[
            pltpu.VMEM((_RPT, _L), jnp.int32),
            pltpu.VMEM((_L, _D), jnp.float32),
            pltpu.VMEM((_L, _D), jnp.float32),
            pltpu.VMEM((80,), jnp.float32),
            pltpu.VMEM((_RPT,), jnp.float32),
            pltpu.SMEM((_RPT,), jnp.float32),
            pltpu.SemaphoreType.DMA,
            pltpu.SemaphoreType.DMA,
        ],
    )
    return f(idx, wb, table)


def kernel(input_ids, emb_table, W, b):
    wb = jnp.concatenate(
        [W.reshape(-1), b.reshape(-1),
         jnp.zeros((15,), jnp.float32)]).astype(jnp.float32)
    out = _run(input_ids, wb, emb_table)
    return out.reshape(_B, 1)


# speedup vs baseline: 6.8509x; 1.0009x over previous
import functools

import jax
import jax.numpy as jnp
from jax import lax
from jax.experimental import pallas as pl
from jax.experimental.pallas import tpu as pltpu
from jax.experimental.pallas import tpu_sc as plsc

_B = 16384
_L = 200
_D = 64
_V = 1000000
_NC = 2
_NS = 16
_NW = _NC * _NS
_RPT = _B // _NW
_SPLIT = 128
_REST = _L - _SPLIT
_TC = 4096
_NBLK = 122
_A = _NBLK * _TC
_TAILB = 640
_PR = _A + _TAILB


def _sigmoid(x):
    return 1.0 / (1.0 + jnp.exp(-x))


def _tr_body(xlo_ref, xhi_ref, o_ref):
    o_ref[:, 0:_D] = xlo_ref[...].T
    o_ref[:, _D:2 * _D] = xhi_ref[...].T


def _tail_body(x_ref, alias_ref, o_ref):
    del alias_ref
    o_ref[:, 0:_D] = jnp.zeros((128, _D), jnp.float32)
    o_ref[:, _D:2 * _D] = x_ref[...].T


def _to_row_major(table):
    tpairs = pl.pallas_call(
        _tr_body,
        grid=(_NBLK,),
        in_specs=[pl.BlockSpec((_D, _TC), lambda i: (0, i)),
                  pl.BlockSpec((_D, _TC), lambda i: (0, i + _NBLK))],
        out_specs=pl.BlockSpec((_TC, 2 * _D), lambda i: (i, 0)),
        out_shape=jax.ShapeDtypeStruct((_PR, 2 * _D), jnp.float32),
    )(table.T, table.T)
    tpairs = pl.pallas_call(
        _tail_body,
        grid=(_TAILB // 128,),
        in_specs=[pl.BlockSpec((_D, 128), lambda i: (0, 2 * _A // 128 + i)),
                  pl.BlockSpec(memory_space=pl.ANY)],
        out_specs=pl.BlockSpec((128, 2 * _D), lambda i: (_A // 128 + i, 0)),
        out_shape=jax.ShapeDtypeStruct((_PR, 2 * _D), jnp.float32),
        input_output_aliases={1: 0},
    )(table.T, tpairs)
    return tpairs.reshape(2 * _PR, _D)


def _remap_body(i_ref, o_ref):
    v = i_ref[...]
    o_ref[...] = jnp.where(v < _A, 2 * v, 2 * (v - _A) + 1)


def _remap_ids(ids):
    blk = 2048
    return pl.pallas_call(
        _remap_body,
        grid=(_B // blk,),
        in_specs=[pl.BlockSpec((blk, _L), lambda i: (i, 0))],
        out_specs=pl.BlockSpec((blk, _L), lambda i: (i, 0)),
        out_shape=jax.ShapeDtypeStruct((_B, _L), jnp.int32),
    )(ids)


def _body(idx_hbm, wb_hbm, table_hbm, out_hbm, idx_v, buf0, buf1, wb_v,
          out_v, out_smem, sem0, sem1):
    wid = lax.axis_index("s") * _NC + lax.axis_index("c")
    base = wid * _RPT

    pltpu.sync_copy(wb_hbm, wb_v)
    pltpu.sync_copy(idx_hbm.at[pl.ds(base, _RPT)], idx_v)

    w0 = wb_v[pl.ds(0, 16)]
    w1 = wb_v[pl.ds(16, 16)]
    w2 = wb_v[pl.ds(32, 16)]
    w3 = wb_v[pl.ds(48, 16)]
    bvec = wb_v[pl.ds(64, 16)]

    def fire(r, buf, sem):
        pltpu.async_copy(
            table_hbm.at[idx_v.at[r, pl.ds(0, _SPLIT)]],
            buf.at[pl.ds(0, _SPLIT)], sem)
        pltpu.async_copy(
            table_hbm.at[idx_v.at[r, pl.ds(_SPLIT, _REST)]],
            buf.at[pl.ds(_SPLIT, _REST)], sem)

    def wait(buf, sem):
        pltpu.make_async_copy(table_hbm.at[pl.ds(0, _L)], buf, sem).wait()

    def process(r, buf):
        def red(j, accs):
            a0, a1, a2, a3, b0, b1, b2, b3 = accs
            row = buf.at[j]
            row2 = buf.at[j + _L // 2]
            return (a0 + row[pl.ds(0, 16)],
                    a1 + row[pl.ds(16, 16)],
                    a2 + row[pl.ds(32, 16)],
                    a3 + row[pl.ds(48, 16)],
                    b0 + row2[pl.ds(0, 16)],
                    b1 + row2[pl.ds(16, 16)],
                    b2 + row2[pl.ds(32, 16)],
                    b3 + row2[pl.ds(48, 16)])

        z = jnp.zeros((16,), jnp.float32)
        a0, a1, a2, a3, b0, b1, b2, b3 = lax.fori_loop(
            0, _L // 2, red, (z, z, z, z, z, z, z, z), unroll=8)
        a0, a1, a2, a3 = a0 + b0, a1 + b1, a2 + b2, a3 + b3
        inv = jnp.float32(1.0 / _L)
        h0 = _sigmoid(a0 * inv)
        h1 = _sigmoid(a1 * inv)
        h2 = _sigmoid(a2 * inv)
        h3 = _sigmoid(a3 * inv)
        t = h0 * w0 + h1 * w1 + h2 * w2 + h3 * w3 + bvec
        out_smem[r] = jnp.sum(t)

    fire(0, buf0, sem0)

    def loop(i, carry):
        r0 = 2 * i
        fire(r0 + 1, buf1, sem1)
        wait(buf0, sem0)
        process(r0, buf0)

        @pl.when(r0 + 2 < _RPT)
        def _():
            fire(r0 + 2, buf0, sem0)

        wait(buf1, sem1)
        process(r0 + 1, buf1)
        return carry

    lax.fori_loop(0, _RPT // 2, loop, 0)

    lane = lax.iota(jnp.int32, 16)

    def pack(g, carry):
        def ins(k, v):
            return jnp.where(lane == k, out_smem[g * 16 + k], v)

        v = lax.fori_loop(0, 16, ins, jnp.zeros((16,), jnp.float32))
        out_v[pl.ds(g * 16, 16)] = _sigmoid(v)
        return carry

    lax.fori_loop(0, _RPT // 16, pack, 0)

    pltpu.sync_copy(out_v, out_hbm.at[pl.ds(base, _RPT)])


@jax.jit
def _run(idx, wb, table):
    table = _to_row_major(table)
    idx = _remap_ids(idx)
    mesh = plsc.VectorSubcoreMesh(core_axis_name="c", subcore_axis_name="s")
    f = pl.kernel(
        _body,
        out_type=jax.ShapeDtypeStruct((_B,), jnp.float32),
        mesh=mesh,
        compiler_params=pltpu.CompilerParams(
            needs_layout_passes=False, use_tc_tiling_on_sc=False),
        scratch_types=[
            pltpu.VMEM((_RPT, _L), jnp.int32),
            pltpu.VMEM((_L, _D), jnp.float32),
            pltpu.VMEM((_L, _D), jnp.float32),
            pltpu.VMEM((80,), jnp.float32),
            pltpu.VMEM((_RPT,), jnp.float32),
            pltpu.SMEM((_RPT,), jnp.float32),
            pltpu.SemaphoreType.DMA,
            pltpu.SemaphoreType.DMA,
        ],
    )
    return f(idx, wb, table)


def kernel(input_ids, emb_table, W, b):
    wb = jnp.concatenate(
        [W.reshape(-1), b.reshape(-1),
         jnp.zeros((15,), jnp.float32)]).astype(jnp.float32)
    out = _run(input_ids, wb, emb_table)
    return out.reshape(_B, 1)


# speedup vs baseline: 7.1840x; 1.0486x over previous
import functools

import jax
import jax.numpy as jnp
from jax import lax
from jax.experimental import pallas as pl
from jax.experimental.pallas import tpu as pltpu
from jax.experimental.pallas import tpu_sc as plsc

_B = 16384
_L = 200
_D = 64
_V = 1000000
_NC = 2
_NS = 16
_NW = _NC * _NS
_RPT = _B // _NW
_SPLIT = 128
_REST = _L - _SPLIT
_TC = 8192
_NBLK = 61
_A = _NBLK * _TC
_TAILB = 640
_PR = _A + _TAILB


def _sigmoid(x):
    return 1.0 / (1.0 + jnp.exp(-x))


def _tr_body(xlo_ref, xhi_ref, o_ref):
    o_ref[:, 0:_D] = xlo_ref[...].T
    o_ref[:, _D:2 * _D] = xhi_ref[...].T


def _tail_body(x_ref, alias_ref, o_ref):
    del alias_ref
    o_ref[:, 0:_D] = jnp.zeros((128, _D), jnp.float32)
    o_ref[:, _D:2 * _D] = x_ref[...].T


def _to_row_major(table):
    tpairs = pl.pallas_call(
        _tr_body,
        grid=(_NBLK,),
        in_specs=[pl.BlockSpec((_D, _TC), lambda i: (0, i)),
                  pl.BlockSpec((_D, _TC), lambda i: (0, i + _NBLK))],
        out_specs=pl.BlockSpec((_TC, 2 * _D), lambda i: (i, 0)),
        out_shape=jax.ShapeDtypeStruct((_PR, 2 * _D), jnp.float32),
    )(table.T, table.T)
    tpairs = pl.pallas_call(
        _tail_body,
        grid=(_TAILB // 128,),
        in_specs=[pl.BlockSpec((_D, 128), lambda i: (0, 2 * _A // 128 + i)),
                  pl.BlockSpec(memory_space=pl.ANY)],
        out_specs=pl.BlockSpec((128, 2 * _D), lambda i: (_A // 128 + i, 0)),
        out_shape=jax.ShapeDtypeStruct((_PR, 2 * _D), jnp.float32),
        input_output_aliases={1: 0},
    )(table.T, tpairs)
    return tpairs.reshape(2 * _PR, _D)


def _remap_body(i_ref, o_ref):
    v = i_ref[...]
    o_ref[...] = jnp.where(v < _A, 2 * v, 2 * (v - _A) + 1)


def _remap_ids(ids):
    blk = 2048
    return pl.pallas_call(
        _remap_body,
        grid=(_B // blk,),
        in_specs=[pl.BlockSpec((blk, _L), lambda i: (i, 0))],
        out_specs=pl.BlockSpec((blk, _L), lambda i: (i, 0)),
        out_shape=jax.ShapeDtypeStruct((_B, _L), jnp.int32),
    )(ids)


def _body(idx_hbm, wb_hbm, table_hbm, out_hbm, idx_v, buf0, buf1, wb_v,
          out_v, out_smem, sem0, sem1):
    wid = lax.axis_index("s") * _NC + lax.axis_index("c")
    base = wid * _RPT

    pltpu.sync_copy(wb_hbm, wb_v)
    pltpu.sync_copy(idx_hbm.at[pl.ds(base, _RPT)], idx_v)

    w0 = wb_v[pl.ds(0, 16)]
    w1 = wb_v[pl.ds(16, 16)]
    w2 = wb_v[pl.ds(32, 16)]
    w3 = wb_v[pl.ds(48, 16)]
    bvec = wb_v[pl.ds(64, 16)]

    def fire(r, buf, sem):
        pltpu.async_copy(
            table_hbm.at[idx_v.at[r, pl.ds(0, _SPLIT)]],
            buf.at[pl.ds(0, _SPLIT)], sem)
        pltpu.async_copy(
            table_hbm.at[idx_v.at[r, pl.ds(_SPLIT, _REST)]],
            buf.at[pl.ds(_SPLIT, _REST)], sem)

    def wait(buf, sem):
        pltpu.make_async_copy(table_hbm.at[pl.ds(0, _L)], buf, sem).wait()

    def process(r, buf):
        def red(j, accs):
            a0, a1, a2, a3, b0, b1, b2, b3 = accs
            row = buf.at[j]
            row2 = buf.at[j + _L // 2]
            return (a0 + row[pl.ds(0, 16)],
                    a1 + row[pl.ds(16, 16)],
                    a2 + row[pl.ds(32, 16)],
                    a3 + row[pl.ds(48, 16)],
                    b0 + row2[pl.ds(0, 16)],
                    b1 + row2[pl.ds(16, 16)],
                    b2 + row2[pl.ds(32, 16)],
                    b3 + row2[pl.ds(48, 16)])

        z = jnp.zeros((16,), jnp.float32)
        a0, a1, a2, a3, b0, b1, b2, b3 = lax.fori_loop(
            0, _L // 2, red, (z, z, z, z, z, z, z, z), unroll=8)
        a0, a1, a2, a3 = a0 + b0, a1 + b1, a2 + b2, a3 + b3
        inv = jnp.float32(1.0 / _L)
        h0 = _sigmoid(a0 * inv)
        h1 = _sigmoid(a1 * inv)
        h2 = _sigmoid(a2 * inv)
        h3 = _sigmoid(a3 * inv)
        t = h0 * w0 + h1 * w1 + h2 * w2 + h3 * w3 + bvec
        out_smem[r] = jnp.sum(t)

    fire(0, buf0, sem0)

    def loop(i, carry):
        r0 = 2 * i
        fire(r0 + 1, buf1, sem1)
        wait(buf0, sem0)
        process(r0, buf0)

        @pl.when(r0 + 2 < _RPT)
        def _():
            fire(r0 + 2, buf0, sem0)

        wait(buf1, sem1)
        process(r0 + 1, buf1)
        return carry

    lax.fori_loop(0, _RPT // 2, loop, 0)

    lane = lax.iota(jnp.int32, 16)

    def pack(g, carry):
        def ins(k, v):
            return jnp.where(lane == k, out_smem[g * 16 + k], v)

        v = lax.fori_loop(0, 16, ins, jnp.zeros((16,), jnp.float32))
        out_v[pl.ds(g * 16, 16)] = _sigmoid(v)
        return carry

    lax.fori_loop(0, _RPT // 16, pack, 0)

    pltpu.sync_copy(out_v, out_hbm.at[pl.ds(base, _RPT)])


@jax.jit
def _run(idx, wb, table):
    table = _to_row_major(table)
    idx = _remap_ids(idx)
    mesh = plsc.VectorSubcoreMesh(core_axis_name="c", subcore_axis_name="s")
    f = pl.kernel(
        _body,
        out_type=jax.ShapeDtypeStruct((_B,), jnp.float32),
        mesh=mesh,
        compiler_params=pltpu.CompilerParams(
            needs_layout_passes=False, use_tc_tiling_on_sc=False),
        scratch_types=[
            pltpu.VMEM((_RPT, _L), jnp.int32),
            pltpu.VMEM((_L, _D), jnp.float32),
            pltpu.VMEM((_L, _D), jnp.float32),
            pltpu.VMEM((80,), jnp.float32),
            pltpu.VMEM((_RPT,), jnp.float32),
            pltpu.SMEM((_RPT,), jnp.float32),
            pltpu.SemaphoreType.DMA,
            pltpu.SemaphoreType.DMA,
        ],
    )
    return f(idx, wb, table)


def kernel(input_ids, emb_table, W, b):
    wb = jnp.concatenate(
        [W.reshape(-1), b.reshape(-1),
         jnp.zeros((15,), jnp.float32)]).astype(jnp.float32)
    out = _run(input_ids, wb, emb_table)
    return out.reshape(_B, 1)


# speedup vs baseline: 7.1871x; 1.0004x over previous
import functools

import jax
import jax.numpy as jnp
from jax import lax
from jax.experimental import pallas as pl
from jax.experimental.pallas import tpu as pltpu
from jax.experimental.pallas import tpu_sc as plsc

_B = 16384
_L = 200
_D = 64
_V = 1000000
_NC = 2
_NS = 16
_NW = _NC * _NS
_RPT = _B // _NW
_SPLIT = 128
_REST = _L - _SPLIT
_TC = 8192
_NBLK = 61
_A = _NBLK * _TC
_TAILB = 640
_PR = _A + _TAILB


def _sigmoid(x):
    return 1.0 / (1.0 + jnp.exp(-x))


def _tr_body(xlo_ref, xhi_ref, o_ref):
    c = 1024
    for k in range(0, _TC, c):
        o_ref[k:k + c, 0:_D] = xlo_ref[:, k:k + c].T
        o_ref[k:k + c, _D:2 * _D] = xhi_ref[:, k:k + c].T


def _tail_body(x_ref, alias_ref, o_ref):
    del alias_ref
    o_ref[:, 0:_D] = jnp.zeros((128, _D), jnp.float32)
    o_ref[:, _D:2 * _D] = x_ref[...].T


def _to_row_major(table):
    tpairs = pl.pallas_call(
        _tr_body,
        grid=(_NBLK,),
        in_specs=[pl.BlockSpec((_D, _TC), lambda i: (0, i)),
                  pl.BlockSpec((_D, _TC), lambda i: (0, i + _NBLK))],
        out_specs=pl.BlockSpec((_TC, 2 * _D), lambda i: (i, 0)),
        out_shape=jax.ShapeDtypeStruct((_PR, 2 * _D), jnp.float32),
    )(table.T, table.T)
    tpairs = pl.pallas_call(
        _tail_body,
        grid=(_TAILB // 128,),
        in_specs=[pl.BlockSpec((_D, 128), lambda i: (0, 2 * _A // 128 + i)),
                  pl.BlockSpec(memory_space=pl.ANY)],
        out_specs=pl.BlockSpec((128, 2 * _D), lambda i: (_A // 128 + i, 0)),
        out_shape=jax.ShapeDtypeStruct((_PR, 2 * _D), jnp.float32),
        input_output_aliases={1: 0},
    )(table.T, tpairs)
    return tpairs.reshape(2 * _PR, _D)


def _remap_body(i_ref, o_ref):
    v = i_ref[...]
    o_ref[...] = jnp.where(v < _A, 2 * v, 2 * (v - _A) + 1)


def _remap_ids(ids):
    blk = 2048
    return pl.pallas_call(
        _remap_body,
        grid=(_B // blk,),
        in_specs=[pl.BlockSpec((blk, _L), lambda i: (i, 0))],
        out_specs=pl.BlockSpec((blk, _L), lambda i: (i, 0)),
        out_shape=jax.ShapeDtypeStruct((_B, _L), jnp.int32),
    )(ids)


def _body(idx_hbm, wb_hbm, table_hbm, out_hbm, idx_v, buf0, buf1, wb_v,
          out_v, out_smem, sem0, sem1):
    wid = lax.axis_index("s") * _NC + lax.axis_index("c")
    base = wid * _RPT

    pltpu.sync_copy(wb_hbm, wb_v)
    pltpu.sync_copy(idx_hbm.at[pl.ds(base, _RPT)], idx_v)

    w0 = wb_v[pl.ds(0, 16)]
    w1 = wb_v[pl.ds(16, 16)]
    w2 = wb_v[pl.ds(32, 16)]
    w3 = wb_v[pl.ds(48, 16)]
    bvec = wb_v[pl.ds(64, 16)]

    def fire(r, buf, sem):
        pltpu.async_copy(
            table_hbm.at[idx_v.at[r, pl.ds(0, _SPLIT)]],
            buf.at[pl.ds(0, _SPLIT)], sem)
        pltpu.async_copy(
            table_hbm.at[idx_v.at[r, pl.ds(_SPLIT, _REST)]],
            buf.at[pl.ds(_SPLIT, _REST)], sem)

    def wait(buf, sem):
        pltpu.make_async_copy(table_hbm.at[pl.ds(0, _L)], buf, sem).wait()

    def process(r, buf):
        def red(j, accs):
            a0, a1, a2, a3, b0, b1, b2, b3 = accs
            row = buf.at[j]
            row2 = buf.at[j + _L // 2]
            return (a0 + row[pl.ds(0, 16)],
                    a1 + row[pl.ds(16, 16)],
                    a2 + row[pl.ds(32, 16)],
                    a3 + row[pl.ds(48, 16)],
                    b0 + row2[pl.ds(0, 16)],
                    b1 + row2[pl.ds(16, 16)],
                    b2 + row2[pl.ds(32, 16)],
                    b3 + row2[pl.ds(48, 16)])

        z = jnp.zeros((16,), jnp.float32)
        a0, a1, a2, a3, b0, b1, b2, b3 = lax.fori_loop(
            0, _L // 2, red, (z, z, z, z, z, z, z, z), unroll=8)
        a0, a1, a2, a3 = a0 + b0, a1 + b1, a2 + b2, a3 + b3
        inv = jnp.float32(1.0 / _L)
        h0 = _sigmoid(a0 * inv)
        h1 = _sigmoid(a1 * inv)
        h2 = _sigmoid(a2 * inv)
        h3 = _sigmoid(a3 * inv)
        t = h0 * w0 + h1 * w1 + h2 * w2 + h3 * w3 + bvec
        out_smem[r] = jnp.sum(t)

    fire(0, buf0, sem0)

    def loop(i, carry):
        r0 = 2 * i
        fire(r0 + 1, buf1, sem1)
        wait(buf0, sem0)
        process(r0, buf0)

        @pl.when(r0 + 2 < _RPT)
        def _():
            fire(r0 + 2, buf0, sem0)

        wait(buf1, sem1)
        process(r0 + 1, buf1)
        return carry

    lax.fori_loop(0, _RPT // 2, loop, 0)

    lane = lax.iota(jnp.int32, 16)

    def pack(g, carry):
        def ins(k, v):
            return jnp.where(lane == k, out_smem[g * 16 + k], v)

        v = lax.fori_loop(0, 16, ins, jnp.zeros((16,), jnp.float32))
        out_v[pl.ds(g * 16, 16)] = _sigmoid(v)
        return carry

    lax.fori_loop(0, _RPT // 16, pack, 0)

    pltpu.sync_copy(out_v, out_hbm.at[pl.ds(base, _RPT)])


@jax.jit
def _run(idx, wb, table):
    table = _to_row_major(table)
    idx = _remap_ids(idx)
    mesh = plsc.VectorSubcoreMesh(core_axis_name="c", subcore_axis_name="s")
    f = pl.kernel(
        _body,
        out_type=jax.ShapeDtypeStruct((_B,), jnp.float32),
        mesh=mesh,
        compiler_params=pltpu.CompilerParams(
            needs_layout_passes=False, use_tc_tiling_on_sc=False),
        scratch_types=[
            pltpu.VMEM((_RPT, _L), jnp.int32),
            pltpu.VMEM((_L, _D), jnp.float32),
            pltpu.VMEM((_L, _D), jnp.float32),
            pltpu.VMEM((80,), jnp.float32),
            pltpu.VMEM((_RPT,), jnp.float32),
            pltpu.SMEM((_RPT,), jnp.float32),
            pltpu.SemaphoreType.DMA,
            pltpu.SemaphoreType.DMA,
        ],
    )
    return f(idx, wb, table)


def kernel(input_ids, emb_table, W, b):
    wb = jnp.concatenate(
        [W.reshape(-1), b.reshape(-1),
         jnp.zeros((15,), jnp.float32)]).astype(jnp.float32)
    out = _run(input_ids, wb, emb_table)
    return out.reshape(_B, 1)


# speedup vs baseline: 7.1881x; 1.0001x over previous
import jax
import jax.numpy as jnp
from jax import lax
from jax.experimental import pallas as pl
from jax.experimental.pallas import tpu as pltpu
from jax.experimental.pallas import tpu_sc as plsc

_B = 16384
_L = 200
_D = 64
_V = 1000000
_NC = 2
_NS = 16
_NW = _NC * _NS
_RPT = _B // _NW
_SPLIT = 128
_REST = _L - _SPLIT
_TC = 8192
_NBLK = 61
_A = _NBLK * _TC
_TAILB = 640
_PR = _A + _TAILB


def _sigmoid(x):
    return 1.0 / (1.0 + jnp.exp(-x))


def _tr_body(xlo_ref, xhi_ref, o_ref):
    c = 1024
    for k in range(0, _TC, c):
        o_ref[k:k + c, 0:_D] = xlo_ref[:, k:k + c].T
        o_ref[k:k + c, _D:2 * _D] = xhi_ref[:, k:k + c].T


def _tail_body(x_ref, alias_ref, o_ref):
    del alias_ref
    o_ref[:, 0:_D] = jnp.zeros((128, _D), jnp.float32)
    o_ref[:, _D:2 * _D] = x_ref[...].T


def _to_row_major(table):
    tpairs = pl.pallas_call(
        _tr_body,
        grid=(_NBLK,),
        in_specs=[pl.BlockSpec((_D, _TC), lambda i: (0, i)),
                  pl.BlockSpec((_D, _TC), lambda i: (0, i + _NBLK))],
        out_specs=pl.BlockSpec((_TC, 2 * _D), lambda i: (i, 0)),
        out_shape=jax.ShapeDtypeStruct((_PR, 2 * _D), jnp.float32),
    )(table.T, table.T)
    tpairs = pl.pallas_call(
        _tail_body,
        grid=(_TAILB // 128,),
        in_specs=[pl.BlockSpec((_D, 128), lambda i: (0, 2 * _A // 128 + i)),
                  pl.BlockSpec(memory_space=pl.ANY)],
        out_specs=pl.BlockSpec((128, 2 * _D), lambda i: (_A // 128 + i, 0)),
        out_shape=jax.ShapeDtypeStruct((_PR, 2 * _D), jnp.float32),
        input_output_aliases={1: 0},
    )(table.T, tpairs)
    return tpairs.reshape(2 * _PR, _D)


def _remap_body(i_ref, o_ref):
    v = i_ref[...]
    o_ref[...] = jnp.where(v < _A, 2 * v, 2 * (v - _A) + 1)


def _remap_ids(ids):
    blk = 2048
    return pl.pallas_call(
        _remap_body,
        grid=(_B // blk,),
        in_specs=[pl.BlockSpec((blk, _L), lambda i: (i, 0))],
        out_specs=pl.BlockSpec((blk, _L), lambda i: (i, 0)),
        out_shape=jax.ShapeDtypeStruct((_B, _L), jnp.int32),
    )(ids)


def _body(idx_hbm, wb_hbm, table_hbm, out_hbm, idx_v, buf0, buf1, wb_v,
          out_v, out_smem, sem0, sem1):
    wid = lax.axis_index("s") * _NC + lax.axis_index("c")
    base = wid * _RPT

    pltpu.sync_copy(wb_hbm, wb_v)
    pltpu.sync_copy(idx_hbm.at[pl.ds(base, _RPT)], idx_v)

    w0 = wb_v[pl.ds(0, 16)]
    w1 = wb_v[pl.ds(16, 16)]
    w2 = wb_v[pl.ds(32, 16)]
    w3 = wb_v[pl.ds(48, 16)]
    bvec = wb_v[pl.ds(64, 16)]

    def fire(r, buf, sem):
        pltpu.async_copy(
            table_hbm.at[idx_v.at[r, pl.ds(0, _SPLIT)]],
            buf.at[pl.ds(0, _SPLIT)], sem)
        pltpu.async_copy(
            table_hbm.at[idx_v.at[r, pl.ds(_SPLIT, _REST)]],
            buf.at[pl.ds(_SPLIT, _REST)], sem)

    def wait(buf, sem):
        pltpu.make_async_copy(table_hbm.at[pl.ds(0, _L)], buf, sem).wait()

    def process(r, buf):
        def red(j, accs):
            a0, a1, a2, a3, b0, b1, b2, b3 = accs
            row = buf.at[j]
            row2 = buf.at[j + _L // 2]
            return (a0 + row[pl.ds(0, 16)],
                    a1 + row[pl.ds(16, 16)],
                    a2 + row[pl.ds(32, 16)],
                    a3 + row[pl.ds(48, 16)],
                    b0 + row2[pl.ds(0, 16)],
                    b1 + row2[pl.ds(16, 16)],
                    b2 + row2[pl.ds(32, 16)],
                    b3 + row2[pl.ds(48, 16)])

        z = jnp.zeros((16,), jnp.float32)
        a0, a1, a2, a3, b0, b1, b2, b3 = lax.fori_loop(
            0, _L // 2, red, (z, z, z, z, z, z, z, z), unroll=8)
        a0, a1, a2, a3 = a0 + b0, a1 + b1, a2 + b2, a3 + b3
        inv = jnp.float32(1.0 / _L)
        h0 = _sigmoid(a0 * inv)
        h1 = _sigmoid(a1 * inv)
        h2 = _sigmoid(a2 * inv)
        h3 = _sigmoid(a3 * inv)
        t = h0 * w0 + h1 * w1 + h2 * w2 + h3 * w3 + bvec
        out_smem[r] = jnp.sum(t)

    fire(0, buf0, sem0)

    def loop(i, carry):
        r0 = 2 * i
        fire(r0 + 1, buf1, sem1)
        wait(buf0, sem0)
        process(r0, buf0)

        @pl.when(r0 + 2 < _RPT)
        def _():
            fire(r0 + 2, buf0, sem0)

        wait(buf1, sem1)
        process(r0 + 1, buf1)
        return carry

    lax.fori_loop(0, _RPT // 2, loop, 0)

    lane = lax.iota(jnp.int32, 16)

    def pack(g, carry):
        def ins(k, v):
            return jnp.where(lane == k, out_smem[g * 16 + k], v)

        v = lax.fori_loop(0, 16, ins, jnp.zeros((16,), jnp.float32))
        out_v[pl.ds(g * 16, 16)] = _sigmoid(v)
        return carry

    lax.fori_loop(0, _RPT // 16, pack, 0)

    pltpu.sync_copy(out_v, out_hbm.at[pl.ds(base, _RPT)])


@jax.jit
def _run(idx, wb, table):
    table = _to_row_major(table)
    idx = _remap_ids(idx)
    mesh = plsc.VectorSubcoreMesh(core_axis_name="c", subcore_axis_name="s")
    f = pl.kernel(
        _body,
        out_type=jax.ShapeDtypeStruct((_B,), jnp.float32),
        mesh=mesh,
        compiler_params=pltpu.CompilerParams(
            needs_layout_passes=False, use_tc_tiling_on_sc=False),
        scratch_types=[
            pltpu.VMEM((_RPT, _L), jnp.int32),
            pltpu.VMEM((_L, _D), jnp.float32),
            pltpu.VMEM((_L, _D), jnp.float32),
            pltpu.VMEM((80,), jnp.float32),
            pltpu.VMEM((_RPT,), jnp.float32),
            pltpu.SMEM((_RPT,), jnp.float32),
            pltpu.SemaphoreType.DMA,
            pltpu.SemaphoreType.DMA,
        ],
    )
    return f(idx, wb, table)


def kernel(input_ids, emb_table, W, b):
    wb = jnp.concatenate(
        [W.reshape(-1), b.reshape(-1),
         jnp.zeros((15,), jnp.float32)]).astype(jnp.float32)
    out = _run(input_ids, wb, emb_table)
    return out.reshape(_B, 1)
